# Initial kernel scaffold; baseline (speedup 1.0000x reference)
import functools

import jax
import jax.numpy as jnp
from jax import lax
from jax.experimental import pallas as pl
from jax.experimental.pallas import tpu as pltpu
from jax.experimental.pallas import tpu_sc as plsc

N = 10000
D = 128
G = 64
BN_EPS = 1e-5

NC = 2
NS = 16
NW = NC * NS

K = 128
EC = 79
E_PAD = NW * EC * K
NP = 10240
RPS = NP // NS
ZR = 32


def _scatter_mesh():
    return plsc.VectorSubcoreMesh(core_axis_name="c", subcore_axis_name="s",
                                  num_cores=NC, num_subcores=NS)


@functools.partial(
    pl.kernel,
    out_type=jax.ShapeDtypeStruct((NC, NP, D), jnp.float32),
    mesh=_scatter_mesh(),
    scratch_types=[
        pltpu.VMEM((EC, K), jnp.int32),
        pltpu.VMEM((EC, K), jnp.int32),
        pltpu.VMEM((K, D), jnp.float32),
        pltpu.VMEM((ZR, D), jnp.float32),
        pltpu.VMEM_SHARED((NP, D), jnp.float32),
        pltpu.SemaphoreType.DMA,
    ],
)
def _sc_scatter(h_hbm, row_hbm, col_hbm, out_hbm,
                row_v, col_v, rows_v, zero_v, agg_sh, sem):
    cid = lax.axis_index("c")
    sid = lax.axis_index("s")
    wid = sid * NC + cid

    def _zfill(i, _):
        zero_v[i // 8, pl.ds((i % 8) * 16, 16)] = jnp.zeros((16,), jnp.float32)
        return ()
    lax.fori_loop(0, ZR * 8, _zfill, ())

    def _zinit(j, _):
        pltpu.sync_copy(zero_v, agg_sh.at[pl.ds(sid * RPS + j * ZR, ZR)])
        return ()
    lax.fori_loop(0, RPS // ZR, _zinit, ())
    plsc.subcore_barrier()

    pltpu.sync_copy(row_hbm.at[wid], row_v)
    pltpu.sync_copy(col_hbm.at[wid], col_v)

    def _chunk(c, _):
        pltpu.async_copy(h_hbm.at[col_v.at[c]], rows_v, sem).wait()
        pltpu.sync_copy(rows_v, agg_sh.at[row_v.at[c]], add=True)
        return ()
    lax.fori_loop(0, EC, _chunk, ())
    plsc.subcore_barrier()

    pltpu.sync_copy(agg_sh.at[pl.ds(sid * RPS, RPS)],
                    out_hbm.at[cid, pl.ds(sid * RPS, RPS)])


def _lin1_body(x_ref, w_ref, b_ref, o_ref):
    h = jnp.dot(x_ref[...], w_ref[...],
                preferred_element_type=jnp.float32,
                precision=lax.Precision.HIGHEST) + b_ref[...]
    o_ref[...] = h * jax.nn.sigmoid(h)


def _combine_body(h_ref, agg_ref, batch_ref, eps_ref,
                  w1_ref, b1_ref, w2_ref, b2_ref,
                  fw1_ref, fb1_ref, fw2_ref, fb2_ref, fw3_ref, fb3_ref,
                  o_ref, acc_ref):
    i = pl.program_id(0)
    t = (1.0 + eps_ref[0, 0]) * h_ref[...] + agg_ref[0] + agg_ref[1]
    r = jnp.maximum(
        jnp.dot(t, w1_ref[...], preferred_element_type=jnp.float32,
                precision=lax.Precision.HIGHEST) + b1_ref[...], 0.0)
    xg = jnp.dot(r, w2_ref[...], preferred_element_type=jnp.float32,
                 precision=lax.Precision.HIGHEST) + b2_ref[...]
    b = batch_ref[0, 0, :]
    onehot = (b[:, None] == lax.broadcasted_iota(jnp.int32, (1, G), 1)
              ).astype(jnp.float32)
    seg = lax.dot_general(onehot, xg, (((0,), (0,)), ((), ())),
                          preferred_element_type=jnp.float32,
                          precision=lax.Precision.HIGHEST)

    @pl.when(i == 0)
    def _():
        acc_ref[...] = jnp.zeros_like(acc_ref)

    acc = acc_ref[...] + seg
    acc_ref[...] = acc

    @pl.when(i == pl.num_programs(0) - 1)
    def _():
        def leaky(v):
            return jnp.where(v >= 0, v, 0.01 * v)
        z = leaky(jnp.dot(acc, fw1_ref[...], preferred_element_type=jnp.float32,
                          precision=lax.Precision.HIGHEST) + fb1_ref[...])
        z = leaky(jnp.dot(z, fw2_ref[...], preferred_element_type=jnp.float32,
                          precision=lax.Precision.HIGHEST) + fb2_ref[...])
        o = lax.dot_general(fw3_ref[...], z, (((1,), (1,)), ((), ())),
                            preferred_element_type=jnp.float32,
                            precision=lax.Precision.HIGHEST)
        o_ref[...] = o + fb3_ref[...]


def kernel(x, pos, edge_index_intra, edge_index_inter, batch, params):
    del pos
    f32 = jnp.float32
    RB = 1000
    grid = N // RB

    h = pl.pallas_call(
        _lin1_body,
        grid=(grid,),
        in_specs=[
            pl.BlockSpec((RB, D), lambda i: (i, 0)),
            pl.BlockSpec((D, D), lambda i: (0, 0)),
            pl.BlockSpec((1, D), lambda i: (0, 0)),
        ],
        out_specs=pl.BlockSpec((RB, D), lambda i: (i, 0)),
        out_shape=jax.ShapeDtypeStruct((N, D), f32),
    )(x, params["lin1_w"], params["lin1_b"].reshape(1, D))

    row = jnp.concatenate([edge_index_intra[0], edge_index_inter[0]])
    col = jnp.concatenate([edge_index_intra[1], edge_index_inter[1]])
    pad = E_PAD - row.shape[0]
    row3 = jnp.concatenate([row, jnp.full((pad,), N, jnp.int32)]
                           ).reshape(NW, EC, K)
    col3 = jnp.concatenate([col, jnp.zeros((pad,), jnp.int32)]
                           ).reshape(NW, EC, K)
    agg2 = _sc_scatter(h, row3, col3)

    lp = params["gin"][-1]
    fc = params["fc"]
    s1 = fc["g1"] / jnp.sqrt(1.0 + BN_EPS)
    s2 = fc["g2"] / jnp.sqrt(1.0 + BN_EPS)
    fw2 = s1[:, None] * fc["w2"]
    fb2 = (fc["be1"] @ fc["w2"] + fc["b2"]).reshape(1, D)
    fw3 = (s2[:, None] * fc["w3"]).reshape(1, D)
    fb3 = (fc["be2"] @ fc["w3"] + fc["b3"]).reshape(1, 1)

    out = pl.pallas_call(
        _combine_body,
        grid=(grid,),
        in_specs=[
            pl.BlockSpec((RB, D), lambda i: (i, 0)),
            pl.BlockSpec((NC, RB, D), lambda i: (0, i, 0)),
            pl.BlockSpec((1, 1, RB), lambda i: (i, 0, 0)),
            pl.BlockSpec((1, 1), lambda i: (0, 0)),
            pl.BlockSpec((D, D), lambda i: (0, 0)),
            pl.BlockSpec((1, D), lambda i: (0, 0)),
            pl.BlockSpec((D, D), lambda i: (0, 0)),
            pl.BlockSpec((1, D), lambda i: (0, 0)),
            pl.BlockSpec((D, D), lambda i: (0, 0)),
            pl.BlockSpec((1, D), lambda i: (0, 0)),
            pl.BlockSpec((D, D), lambda i: (0, 0)),
            pl.BlockSpec((1, D), lambda i: (0, 0)),
            pl.BlockSpec((1, D), lambda i: (0, 0)),
            pl.BlockSpec((1, 1), lambda i: (0, 0)),
        ],
        out_specs=pl.BlockSpec((1, G), lambda i: (0, 0)),
        out_shape=jax.ShapeDtypeStruct((1, G), f32),
        scratch_shapes=[pltpu.VMEM((G, D), f32)],
    )(h, agg2, batch.reshape(grid, 1, RB), lp["eps"].reshape(1, 1),
      lp["w1"], lp["b1"].reshape(1, D), lp["w2"], lp["b2"].reshape(1, D),
      fc["w1"], fc["b1"].reshape(1, D), fw2, fb2, fw3, fb3)
    return out.reshape(-1)


# speedup vs baseline: 4.6428x; 4.6428x over previous
import functools

import jax
import jax.numpy as jnp
from jax import lax
from jax.experimental import pallas as pl
from jax.experimental.pallas import tpu as pltpu
from jax.experimental.pallas import tpu_sc as plsc

N = 10000
D = 128
G = 64
BN_EPS = 1e-5

NC = 2
NS = 16
NW = NC * NS

K = 128
EC = 79
E_PAD = NW * EC * K
NP = 10240
RPS = NP // NS
ZR = 32


@functools.cache
def _make_sc_scatter():
    mesh = plsc.VectorSubcoreMesh(core_axis_name="c", subcore_axis_name="s",
                                  num_cores=NC, num_subcores=NS)

    @functools.partial(
        pl.kernel,
        out_type=jax.ShapeDtypeStruct((NC, NP, D), jnp.float32),
        mesh=mesh,
        scratch_types=[
            pltpu.VMEM((EC, K), jnp.int32),
            pltpu.VMEM((EC, K), jnp.int32),
            pltpu.VMEM((K, D), jnp.float32),
            pltpu.VMEM((ZR, D), jnp.float32),
            pltpu.VMEM_SHARED((NP, D), jnp.float32),
            pltpu.SemaphoreType.DMA,
        ],
    )
    def _sc_scatter(h_hbm, row_hbm, col_hbm, out_hbm,
                    row_v, col_v, rows_v, zero_v, agg_sh, sem):
        cid = lax.axis_index("c")
        sid = lax.axis_index("s")
        wid = sid * NC + cid

        def _zfill(i, _):
            zero_v[i // 8, pl.ds((i % 8) * 16, 16)] = jnp.zeros((16,),
                                                                jnp.float32)
            return ()
        lax.fori_loop(0, ZR * 8, _zfill, ())

        def _zinit(j, _):
            pltpu.sync_copy(zero_v, agg_sh.at[pl.ds(sid * RPS + j * ZR, ZR)])
            return ()
        lax.fori_loop(0, RPS // ZR, _zinit, ())
        plsc.subcore_barrier()

        pltpu.sync_copy(row_hbm.at[wid], row_v)
        pltpu.sync_copy(col_hbm.at[wid], col_v)

        def _chunk(c, _):
            pltpu.async_copy(h_hbm.at[col_v.at[c]], rows_v, sem).wait()
            pltpu.sync_copy(rows_v, agg_sh.at[row_v.at[c]], add=True)
            return ()
        lax.fori_loop(0, EC, _chunk, ())
        plsc.subcore_barrier()

        pltpu.sync_copy(agg_sh.at[pl.ds(sid * RPS, RPS)],
                        out_hbm.at[cid, pl.ds(sid * RPS, RPS)])

    return _sc_scatter


def _lin1_body(x_ref, w_ref, b_ref, o_ref):
    h = jnp.dot(x_ref[...], w_ref[...],
                preferred_element_type=jnp.float32,
                precision=lax.Precision.HIGHEST) + b_ref[...]
    o_ref[...] = h * jax.nn.sigmoid(h)


def _combine_body(h_ref, agg_ref, batch_ref, eps_ref,
                  w1_ref, b1_ref, w2_ref, b2_ref,
                  fw1_ref, fb1_ref, fw2_ref, fb2_ref, fw3_ref, fb3_ref,
                  o_ref, acc_ref):
    i = pl.program_id(0)
    t = (1.0 + eps_ref[0, 0]) * h_ref[...] + agg_ref[0] + agg_ref[1]
    r = jnp.maximum(
        jnp.dot(t, w1_ref[...], preferred_element_type=jnp.float32,
                precision=lax.Precision.HIGHEST) + b1_ref[...], 0.0)
    xg = jnp.dot(r, w2_ref[...], preferred_element_type=jnp.float32,
                 precision=lax.Precision.HIGHEST) + b2_ref[...]
    b = batch_ref[0, 0, :]
    onehot = (b[:, None] == lax.broadcasted_iota(jnp.int32, (1, G), 1)
              ).astype(jnp.float32)
    seg = lax.dot_general(onehot, xg, (((0,), (0,)), ((), ())),
                          preferred_element_type=jnp.float32,
                          precision=lax.Precision.HIGHEST)

    @pl.when(i == 0)
    def _():
        acc_ref[...] = jnp.zeros_like(acc_ref)

    acc = acc_ref[...] + seg
    acc_ref[...] = acc

    @pl.when(i == pl.num_programs(0) - 1)
    def _():
        def leaky(v):
            return jnp.where(v >= 0, v, 0.01 * v)
        z = leaky(jnp.dot(acc, fw1_ref[...], preferred_element_type=jnp.float32,
                          precision=lax.Precision.HIGHEST) + fb1_ref[...])
        z = leaky(jnp.dot(z, fw2_ref[...], preferred_element_type=jnp.float32,
                          precision=lax.Precision.HIGHEST) + fb2_ref[...])
        o = lax.dot_general(fw3_ref[...], z, (((1,), (1,)), ((), ())),
                            preferred_element_type=jnp.float32,
                            precision=lax.Precision.HIGHEST)
        o_ref[...] = o + fb3_ref[...]


def kernel(x, pos, edge_index_intra, edge_index_inter, batch, params):
    del pos
    f32 = jnp.float32
    RB = 1000
    grid = N // RB

    h = pl.pallas_call(
        _lin1_body,
        grid=(grid,),
        in_specs=[
            pl.BlockSpec((RB, D), lambda i: (i, 0)),
            pl.BlockSpec((D, D), lambda i: (0, 0)),
            pl.BlockSpec((1, D), lambda i: (0, 0)),
        ],
        out_specs=pl.BlockSpec((RB, D), lambda i: (i, 0)),
        out_shape=jax.ShapeDtypeStruct((N, D), f32),
    )(x, params["lin1_w"], params["lin1_b"].reshape(1, D))

    row = jnp.concatenate([edge_index_intra[0], edge_index_inter[0]])
    col = jnp.concatenate([edge_index_intra[1], edge_index_inter[1]])
    pad = E_PAD - row.shape[0]
    row3 = jnp.concatenate([row, jnp.full((pad,), N, jnp.int32)]
                           ).reshape(NW, EC, K)
    col3 = jnp.concatenate([col, jnp.zeros((pad,), jnp.int32)]
                           ).reshape(NW, EC, K)
    agg2 = _make_sc_scatter()(h, row3, col3)

    lp = params["gin"][-1]
    fc = params["fc"]
    s1 = fc["g1"] / jnp.sqrt(1.0 + BN_EPS)
    s2 = fc["g2"] / jnp.sqrt(1.0 + BN_EPS)
    fw2 = s1[:, None] * fc["w2"]
    fb2 = (fc["be1"] @ fc["w2"] + fc["b2"]).reshape(1, D)
    fw3 = (s2[:, None] * fc["w3"]).reshape(1, D)
    fb3 = (fc["be2"] @ fc["w3"] + fc["b3"]).reshape(1, 1)

    out = pl.pallas_call(
        _combine_body,
        grid=(grid,),
        in_specs=[
            pl.BlockSpec((RB, D), lambda i: (i, 0)),
            pl.BlockSpec((NC, RB, D), lambda i: (0, i, 0)),
            pl.BlockSpec((1, 1, RB), lambda i: (i, 0, 0)),
            pl.BlockSpec((1, 1), lambda i: (0, 0)),
            pl.BlockSpec((D, D), lambda i: (0, 0)),
            pl.BlockSpec((1, D), lambda i: (0, 0)),
            pl.BlockSpec((D, D), lambda i: (0, 0)),
            pl.BlockSpec((1, D), lambda i: (0, 0)),
            pl.BlockSpec((D, D), lambda i: (0, 0)),
            pl.BlockSpec((1, D), lambda i: (0, 0)),
            pl.BlockSpec((D, D), lambda i: (0, 0)),
            pl.BlockSpec((1, D), lambda i: (0, 0)),
            pl.BlockSpec((1, D), lambda i: (0, 0)),
            pl.BlockSpec((1, 1), lambda i: (0, 0)),
        ],
        out_specs=pl.BlockSpec((1, G), lambda i: (0, 0)),
        out_shape=jax.ShapeDtypeStruct((1, G), f32),
        scratch_shapes=[pltpu.VMEM((G, D), f32)],
    )(h, agg2, batch.reshape(grid, 1, RB), lp["eps"].reshape(1, 1),
      lp["w1"], lp["b1"].reshape(1, D), lp["w2"], lp["b2"].reshape(1, D),
      fc["w1"], fc["b1"].reshape(1, D), fw2, fb2, fw3, fb3)
    return out.reshape(-1)
